# baseline (speedup 1.0000x reference)
import dataclasses
import functools
import math

import jax
import jax.numpy as jnp
from jax import lax
from jax.experimental import pallas as pl
from jax.experimental.pallas import tpu as pltpu
from jax.experimental.pallas import tpu_sc as plsc

NUM_OBJ = 1000
NUM_REL = 64
BATCH = 16384
EPS = 1e-3
LOG_PSB = math.log(1e-3)
LOG_BG = math.log(1e-3)

NC, NS, L = 2, 16, 16
NW = NC * NS
TBL = NUM_OBJ * NUM_OBJ
SCW = 2048
NSC = 16
NSC_FULL = 488
SP0 = NSC_FULL * SCW
SPW = 512
TAIL0 = SP0 + SPW
TAILW = TBL - TAIL0
NBLK = NSC * 8
OUT_PAD = BATCH + 1024
STG = 64
STG_SH = 6
NH = 4
LCAP = 2048
MCAP = 192


def _sc_body(op_hbm, fgT_hbm, tail_hbm, out_hbm,
             op_v, f_l, b_l, buf0, buf1, tail_v, stage_v, bring_v,
             mdf_v, mp_v, sem0, sem1, sems):
    wid = lax.axis_index("s") * NC + lax.axis_index("c")
    lanes = lax.iota(jnp.int32, L)
    zeros = lanes * 0

    f_lo = wid * (NSC * SCW)
    f_hi = jnp.minimum(f_lo + NSC * SCW, TBL)

    pltpu.sync_copy(op_hbm, op_v)

    def pre_group(t, nn):
        rows = t * L + lanes
        a = op_v[0, pl.ds(t * L, L)]
        b = op_v[1, pl.ds(t * L, L)]
        fv = a * NUM_OBJ + b
        m = (fv >= f_lo) & (fv < f_hi)
        pos = plsc.cumsum(m.astype(jnp.int32))
        dst = jnp.minimum(nn + pos - 1, LCAP - 1)
        plsc.store_scatter(f_l, [dst], fv, mask=m)
        plsc.store_scatter(b_l, [dst], rows, mask=m)
        return nn + lax.reduce_sum(m.astype(jnp.int32), axes=(0,))

    n = lax.fori_loop(0, BATCH // L, pre_group, jnp.int32(0))
    n = jnp.minimum(n, LCAP)
    ng = (n + L - 1) // L

    def sc_lane0(s):
        return jnp.minimum(wid * NSC + s, NSC_FULL - 1) * SCW

    def phase_a(F, c0, cw):
        def scan_group(g, carry):
            FF, mc = carry
            idxl = g * L + lanes
            fv = plsc.load_gather(f_l, [idxl])
            bv = plsc.load_gather(b_l, [idxl])
            m = (idxl < n) & (fv >= c0) & (fv < c0 + cw)
            pos = plsc.cumsum(m.astype(jnp.int32))
            mok = m & (mc + pos - 1 < MCAP)
            pos = plsc.cumsum(mok.astype(jnp.int32))
            dfv = jnp.where(mok, fv - c0, 0)
            bvs = jnp.where(mok, bv, BATCH)
            P = FF + pos - 1
            mdst = jnp.minimum(mc + pos - 1, MCAP - 1)
            plsc.store_scatter(mdf_v, [mdst], dfv, mask=mok)
            plsc.store_scatter(mp_v, [mdst], P, mask=mok)
            plsc.store_scatter(
                bring_v, [(P >> STG_SH) & (NH - 1), P & (STG - 1)],
                bvs, mask=mok)
            cnt = lax.reduce_sum(mok.astype(jnp.int32), axes=(0,))
            return (FF + cnt, mc + cnt)

        return lax.fori_loop(0, ng, scan_group, (F, jnp.int32(0)))

    def flush(h):
        pltpu.async_copy(
            stage_v.at[h], out_hbm.at[bring_v.at[h]], sems
        ).wait()

    def flush_upto(F, H):
        tgt = F >> STG_SH
        for k in range(NH):
            @pl.when(H + k < tgt)
            def _():
                flush((H + k) & (NH - 1))
        return tgt

    def extract_block(buf, row_base, g8, mc):
        def egroup(q, _):
            idxl = q * L + lanes
            m = idxl < mc
            dfv = plsc.load_gather(mdf_v, [idxl])
            dfv = jnp.where(m, dfv, 0)
            P = plsc.load_gather(mp_v, [idxl])
            hv = (P >> STG_SH) & (NH - 1)
            pv = P & (STG - 1)
            for r8 in range(8):
                vals = plsc.load_gather(buf, [zeros + (row_base + r8), dfv])
                plsc.store_scatter(
                    stage_v, [hv, pv, zeros + r8 + g8 * 8], vals, mask=m)
            return 0

        lax.fori_loop(0, (mc + L - 1) // L, egroup, 0)

    def start(j, buf, sem):
        jj = jnp.minimum(j, NBLK - 1)
        c0 = pl.multiple_of(sc_lane0(jj >> 3), SCW)
        r0 = pl.multiple_of((jj & 7) * 8, 8)
        pltpu.async_copy(
            fgT_hbm.at[pl.ds(r0, 8), pl.ds(c0, SCW)], buf, sem)

    def wait(buf, sem):
        pltpu.make_async_copy(
            fgT_hbm.at[pl.ds(0, 8), pl.ds(0, SCW)], buf, sem).wait()

    def after_block(j, buf, F, H, mc):
        extract_block(buf, 0, j & 7, mc)

        def boundary(args):
            FF, HH, _ = args
            HH = flush_upto(FF, HH)
            FF, mcn = phase_a(FF, sc_lane0((j >> 3) + 1), SCW)
            return (FF, HH, mcn)

        return lax.cond(((j & 7) == 7) & (j < NBLK - 1), boundary,
                        lambda a: a, (F, H, mc))

    F, mc = phase_a(jnp.int32(0), sc_lane0(0), SCW)
    H = jnp.int32(0)
    start(0, buf0, sem0)

    def pipe(jj, carry):
        F, H, mc = carry
        j0 = 2 * jj
        start(j0 + 1, buf1, sem1)
        wait(buf0, sem0)
        F, H, mc = after_block(j0, buf0, F, H, mc)
        start(j0 + 2, buf0, sem0)
        wait(buf1, sem1)
        F, H, mc = after_block(j0 + 1, buf1, F, H, mc)
        return (F, H, mc)

    F, H, mc = lax.fori_loop(0, NBLK // 2, pipe, (F, H, mc))
    wait(buf0, sem0)

    H = flush_upto(F, H)
    F, mc = phase_a(F, jnp.int32(SP0), SPW)
    for g8 in range(8):
        pltpu.sync_copy(
            fgT_hbm.at[pl.ds(g8 * 8, 8), pl.ds(SP0, SPW)],
            buf0.at[:, pl.ds(0, SPW)])
        extract_block(buf0, 0, g8, mc)

    H = flush_upto(F, H)
    F, mc = phase_a(F, jnp.int32(TAIL0), TAILW)
    pltpu.sync_copy(tail_hbm, tail_v)
    for g8 in range(8):
        extract_block(tail_v, g8 * 8, g8, mc)

    H = flush_upto(F, H)
    resid = F & (STG - 1)

    @pl.when(resid > 0)
    def _():
        h = (F >> STG_SH) & (NH - 1)
        for g in range(STG // L):
            p16 = g * L + lanes
            mpad = p16 >= resid
            plsc.store_scatter(bring_v, [zeros + h, p16], BATCH + p16,
                               mask=mpad)
        flush(h)


def _sc_gather(opT, fgT, tailT):
    mesh = plsc.VectorSubcoreMesh(core_axis_name="c", subcore_axis_name="s")
    cp = pltpu.CompilerParams()
    if "needs_layout_passes" in pltpu.CompilerParams.__dataclass_fields__:
        cp = dataclasses.replace(cp, needs_layout_passes=False)
    cp = dataclasses.replace(cp, use_tc_tiling_on_sc=True)
    k = pl.kernel(
        _sc_body,
        out_type=jax.ShapeDtypeStruct((OUT_PAD, 2 * NUM_REL), jnp.float32),
        mesh=mesh,
        scratch_types=[
            pltpu.VMEM((2, BATCH), jnp.int32),
            pltpu.VMEM((LCAP,), jnp.int32),
            pltpu.VMEM((LCAP,), jnp.int32),
            pltpu.VMEM((8, SCW), jnp.float32),
            pltpu.VMEM((8, SCW), jnp.float32),
            pltpu.VMEM((NUM_REL, TAILW), jnp.float32),
            pltpu.VMEM((NH, STG, 2 * NUM_REL), jnp.float32),
            pltpu.VMEM((NH, STG), jnp.int32),
            pltpu.VMEM((MCAP,), jnp.int32),
            pltpu.VMEM((MCAP,), jnp.int32),
            pltpu.SemaphoreType.DMA,
            pltpu.SemaphoreType.DMA,
            pltpu.SemaphoreType.DMA,
        ],
        compiler_params=cp,
    )
    return k(opT, fgT, tailT)


def _tc_fuse_body(counts_ref, pred_ref, out_ref):
    c = counts_ref[:, :NUM_REL]
    denom = jnp.sum(c, axis=1, keepdims=True) + EPS
    bias = jnp.log(c / denom + EPS)
    bias = jnp.where(c == 0.0, LOG_PSB, bias)
    col = lax.broadcasted_iota(jnp.int32, c.shape, 1)
    bias = jnp.where(col == 0, LOG_BG, bias)
    out_ref[...] = pred_ref[...] + bias


def _tc_fuse(counts_pad, pred_dist):
    blk = 1024
    grid = BATCH // blk
    return pl.pallas_call(
        _tc_fuse_body,
        out_shape=jax.ShapeDtypeStruct((BATCH, NUM_REL), jnp.float32),
        grid=(grid,),
        in_specs=[
            pl.BlockSpec((blk, 2 * NUM_REL), lambda i: (i, 0)),
            pl.BlockSpec((blk, NUM_REL), lambda i: (i, 0)),
        ],
        out_specs=pl.BlockSpec((blk, NUM_REL), lambda i: (i, 0)),
    )(counts_pad, pred_dist)


def kernel(pred_dist, gt, obj_pair, fg_count):
    del gt
    fgT = fg_count.T
    tailT = fgT[:, TAIL0:]
    counts_pad = _sc_gather(obj_pair.T, fgT, tailT)
    return _tc_fuse(counts_pad, pred_dist)


# speedup vs baseline: 1.1372x; 1.1372x over previous
import dataclasses
import functools
import math

import jax
import jax.numpy as jnp
from jax import lax
from jax.experimental import pallas as pl
from jax.experimental.pallas import tpu as pltpu
from jax.experimental.pallas import tpu_sc as plsc

NUM_OBJ = 1000
NUM_REL = 64
BATCH = 16384
EPS = 1e-3
LOG_PSB = math.log(1e-3)
LOG_BG = math.log(1e-3)

NC, NS, L = 2, 16, 16
NW = NC * NS
TBL = NUM_OBJ * NUM_OBJ
SCW = 2048
NSC = 16
NSC_FULL = 488
SP0 = NSC_FULL * SCW
SPW = 512
TAIL0 = SP0 + SPW
TAILW = TBL - TAIL0
NBLK = NSC * 8
OUT_PAD = BATCH + 1024
STG = 64
STG_SH = 6
NH = 4
LCAP = 2048
OPB = 4096
MCAP = 192


def _sc_body(op_hbm, fgT_hbm, tail_hbm, out_hbm,
             op_v, f_l, b_l, buf0, buf1, buf2, buf3, tail_v, stage_v,
             bring_v, mdf_v, mp_v, sem0, sem1, sem2, sem3, sems):
    wid = lax.axis_index("s") * NC + lax.axis_index("c")
    lanes = lax.iota(jnp.int32, L)
    zeros = lanes * 0

    f_lo = wid * (NSC * SCW)
    f_hi = jnp.minimum(f_lo + NSC * SCW, TBL)

    n = jnp.int32(0)
    for oblk in range(BATCH // OPB):
        pltpu.sync_copy(op_hbm.at[:, pl.ds(oblk * OPB, OPB)], op_v)

        def pre_group(t, nn, oblk=oblk):
            rows = (oblk * OPB // L + t) * L + lanes
            a = op_v[0, pl.ds(t * L, L)]
            b = op_v[1, pl.ds(t * L, L)]
            fv = a * NUM_OBJ + b
            m = (fv >= f_lo) & (fv < f_hi)
            pos = plsc.cumsum(m.astype(jnp.int32))
            dst = jnp.minimum(nn + pos - 1, LCAP - 1)
            plsc.store_scatter(f_l, [dst], fv, mask=m)
            plsc.store_scatter(b_l, [dst], rows, mask=m)
            return nn + lax.reduce_sum(m.astype(jnp.int32), axes=(0,))

        n = lax.fori_loop(0, OPB // L, pre_group, n)
    n = jnp.minimum(n, LCAP)
    ng = (n + L - 1) // L

    def sc_lane0(s):
        return jnp.minimum(wid * NSC + s, NSC_FULL - 1) * SCW

    def phase_a(F, c0, cw):
        def scan_group(g, carry):
            FF, mc = carry
            idxl = g * L + lanes
            fv = plsc.load_gather(f_l, [idxl])
            bv = plsc.load_gather(b_l, [idxl])
            m = (idxl < n) & (fv >= c0) & (fv < c0 + cw)
            pos = plsc.cumsum(m.astype(jnp.int32))
            mok = m & (mc + pos - 1 < MCAP)
            pos = plsc.cumsum(mok.astype(jnp.int32))
            dfv = jnp.where(mok, fv - c0, 0)
            bvs = jnp.where(mok, bv, BATCH)
            P = FF + pos - 1
            mdst = jnp.minimum(mc + pos - 1, MCAP - 1)
            plsc.store_scatter(mdf_v, [mdst], dfv, mask=mok)
            plsc.store_scatter(mp_v, [mdst], P, mask=mok)
            plsc.store_scatter(
                bring_v, [(P >> STG_SH) & (NH - 1), P & (STG - 1)],
                bvs, mask=mok)
            cnt = lax.reduce_sum(mok.astype(jnp.int32), axes=(0,))
            return (FF + cnt, mc + cnt)

        return lax.fori_loop(0, ng, scan_group, (F, jnp.int32(0)))

    def flush(h):
        pltpu.async_copy(
            stage_v.at[h], out_hbm.at[bring_v.at[h]], sems
        ).wait()

    def flush_upto(F, H):
        tgt = F >> STG_SH
        for k in range(NH):
            @pl.when(H + k < tgt)
            def _():
                flush((H + k) & (NH - 1))
        return tgt

    def extract_block(buf, row_base, g8, mc):
        def egroup(q, _):
            idxl = q * L + lanes
            m = idxl < mc
            dfv = plsc.load_gather(mdf_v, [idxl])
            dfv = jnp.where(m, dfv, 0)
            P = plsc.load_gather(mp_v, [idxl])
            hv = (P >> STG_SH) & (NH - 1)
            pv = P & (STG - 1)
            for r8 in range(8):
                vals = plsc.load_gather(buf, [zeros + (row_base + r8), dfv])
                plsc.store_scatter(
                    stage_v, [hv, pv, zeros + r8 + g8 * 8], vals, mask=m)
            return 0

        lax.fori_loop(0, (mc + L - 1) // L, egroup, 0)

    def start(j, buf, sem):
        jj = jnp.minimum(j, NBLK - 1)
        c0 = pl.multiple_of(sc_lane0(jj >> 3), SCW)
        r0 = pl.multiple_of((jj & 7) * 8, 8)
        pltpu.async_copy(
            fgT_hbm.at[pl.ds(r0, 8), pl.ds(c0, SCW)], buf, sem)

    def wait(buf, sem):
        pltpu.make_async_copy(
            fgT_hbm.at[pl.ds(0, 8), pl.ds(0, SCW)], buf, sem).wait()

    def after_block(j, buf, F, H, mc):
        extract_block(buf, 0, j & 7, mc)

        def boundary(args):
            FF, HH, _ = args
            HH = flush_upto(FF, HH)
            FF, mcn = phase_a(FF, sc_lane0((j >> 3) + 1), SCW)
            return (FF, HH, mcn)

        return lax.cond(((j & 7) == 7) & (j < NBLK - 1), boundary,
                        lambda a: a, (F, H, mc))

    F, mc = phase_a(jnp.int32(0), sc_lane0(0), SCW)
    H = jnp.int32(0)
    bufs = (buf0, buf1, buf2, buf3)
    bsems = (sem0, sem1, sem2, sem3)
    start(0, buf0, sem0)
    start(1, buf1, sem1)
    start(2, buf2, sem2)

    def pipe(jj, carry):
        F, H, mc = carry
        j0 = 4 * jj
        for t in range(4):
            start(j0 + t + 3, bufs[(t + 3) % 4], bsems[(t + 3) % 4])
            wait(bufs[t], bsems[t])
            F, H, mc = after_block(j0 + t, bufs[t], F, H, mc)
        return (F, H, mc)

    F, H, mc = lax.fori_loop(0, NBLK // 4, pipe, (F, H, mc))
    wait(buf0, sem0)
    wait(buf1, sem1)
    wait(buf2, sem2)

    H = flush_upto(F, H)
    F, mc = phase_a(F, jnp.int32(SP0), SPW)
    for g8 in range(8):
        pltpu.sync_copy(
            fgT_hbm.at[pl.ds(g8 * 8, 8), pl.ds(SP0, SPW)],
            buf0.at[:, pl.ds(0, SPW)])
        extract_block(buf0, 0, g8, mc)

    H = flush_upto(F, H)
    F, mc = phase_a(F, jnp.int32(TAIL0), TAILW)
    pltpu.sync_copy(tail_hbm, tail_v)
    for g8 in range(8):
        extract_block(tail_v, g8 * 8, g8, mc)

    H = flush_upto(F, H)
    resid = F & (STG - 1)

    @pl.when(resid > 0)
    def _():
        h = (F >> STG_SH) & (NH - 1)
        for g in range(STG // L):
            p16 = g * L + lanes
            mpad = p16 >= resid
            plsc.store_scatter(bring_v, [zeros + h, p16], BATCH + p16,
                               mask=mpad)
        flush(h)


def _sc_gather(opT, fgT, tailT):
    mesh = plsc.VectorSubcoreMesh(core_axis_name="c", subcore_axis_name="s")
    cp = pltpu.CompilerParams()
    if "needs_layout_passes" in pltpu.CompilerParams.__dataclass_fields__:
        cp = dataclasses.replace(cp, needs_layout_passes=False)
    cp = dataclasses.replace(cp, use_tc_tiling_on_sc=True)
    k = pl.kernel(
        _sc_body,
        out_type=jax.ShapeDtypeStruct((OUT_PAD, 2 * NUM_REL), jnp.float32),
        mesh=mesh,
        scratch_types=[
            pltpu.VMEM((2, OPB), jnp.int32),
            pltpu.VMEM((LCAP,), jnp.int32),
            pltpu.VMEM((LCAP,), jnp.int32),
            pltpu.VMEM((8, SCW), jnp.float32),
            pltpu.VMEM((8, SCW), jnp.float32),
            pltpu.VMEM((8, SCW), jnp.float32),
            pltpu.VMEM((8, SCW), jnp.float32),
            pltpu.VMEM((NUM_REL, TAILW), jnp.float32),
            pltpu.VMEM((NH, STG, 2 * NUM_REL), jnp.float32),
            pltpu.VMEM((NH, STG), jnp.int32),
            pltpu.VMEM((MCAP,), jnp.int32),
            pltpu.VMEM((MCAP,), jnp.int32),
            pltpu.SemaphoreType.DMA,
            pltpu.SemaphoreType.DMA,
            pltpu.SemaphoreType.DMA,
            pltpu.SemaphoreType.DMA,
            pltpu.SemaphoreType.DMA,
        ],
        compiler_params=cp,
    )
    return k(opT, fgT, tailT)


def _tc_fuse_body(counts_ref, pred_ref, out_ref):
    c = counts_ref[:, :NUM_REL]
    denom = jnp.sum(c, axis=1, keepdims=True) + EPS
    bias = jnp.log(c / denom + EPS)
    bias = jnp.where(c == 0.0, LOG_PSB, bias)
    col = lax.broadcasted_iota(jnp.int32, c.shape, 1)
    bias = jnp.where(col == 0, LOG_BG, bias)
    out_ref[...] = pred_ref[...] + bias


def _tc_fuse(counts_pad, pred_dist):
    blk = 1024
    grid = BATCH // blk
    return pl.pallas_call(
        _tc_fuse_body,
        out_shape=jax.ShapeDtypeStruct((BATCH, NUM_REL), jnp.float32),
        grid=(grid,),
        in_specs=[
            pl.BlockSpec((blk, 2 * NUM_REL), lambda i: (i, 0)),
            pl.BlockSpec((blk, NUM_REL), lambda i: (i, 0)),
        ],
        out_specs=pl.BlockSpec((blk, NUM_REL), lambda i: (i, 0)),
    )(counts_pad, pred_dist)


def kernel(pred_dist, gt, obj_pair, fg_count):
    del gt
    fgT = fg_count.T
    tailT = fgT[:, TAIL0:]
    counts_pad = _sc_gather(obj_pair.T, fgT, tailT)
    return _tc_fuse(counts_pad, pred_dist)


# speedup vs baseline: 1.1948x; 1.0506x over previous
import dataclasses
import functools
import math

import jax
import jax.numpy as jnp
from jax import lax
from jax.experimental import pallas as pl
from jax.experimental.pallas import tpu as pltpu
from jax.experimental.pallas import tpu_sc as plsc

NUM_OBJ = 1000
NUM_REL = 64
BATCH = 16384
EPS = 1e-3
LOG_PSB = math.log(1e-3)
LOG_BG = math.log(1e-3)

NC, NS, L = 2, 16, 16
NW = NC * NS
TBL = NUM_OBJ * NUM_OBJ
SCW = 2048
NSC = 16
NSC_FULL = 488
SP0 = NSC_FULL * SCW
SPW = 512
TAIL0 = SP0 + SPW
TAILW = TBL - TAIL0
NBLK = NSC * 8
OUT_PAD = BATCH + 1024
STG = 64
STG_SH = 6
NH = 4
LCAP = 2048
OPB = 4096
MCAP = 192


def _sc_body(op_hbm, fgT_hbm, tail_hbm, out_hbm,
             op_v, f_l, b_l, buf0, buf1, buf2, buf3, tail_v, stage_v,
             bring_v, mdf_v, mp_v, sem0, sem1, sem2, sem3, sems):
    wid = lax.axis_index("s") * NC + lax.axis_index("c")
    lanes = lax.iota(jnp.int32, L)
    zeros = lanes * 0

    f_lo = wid * (NSC * SCW)
    f_hi = jnp.minimum(f_lo + NSC * SCW, TBL)

    n = jnp.int32(0)
    for oblk in range(BATCH // OPB):
        pltpu.sync_copy(op_hbm.at[:, pl.ds(oblk * OPB, OPB)], op_v)

        def pre_group(t, nn, oblk=oblk):
            rows = (oblk * OPB // L + t) * L + lanes
            a = op_v[0, pl.ds(t * L, L)]
            b = op_v[1, pl.ds(t * L, L)]
            fv = a * NUM_OBJ + b
            m = (fv >= f_lo) & (fv < f_hi)
            pos = plsc.cumsum(m.astype(jnp.int32))
            dst = jnp.minimum(nn + pos - 1, LCAP - 1)
            plsc.store_scatter(f_l, [dst], fv, mask=m)
            plsc.store_scatter(b_l, [dst], rows, mask=m)
            return nn + lax.reduce_sum(m.astype(jnp.int32), axes=(0,))

        n = lax.fori_loop(0, OPB // L, pre_group, n)
    n = jnp.minimum(n, LCAP)
    ng = (n + L - 1) // L

    def sc_lane0(s):
        return jnp.minimum(wid * NSC + s, NSC_FULL - 1) * SCW

    def phase_a(F, c0, cw):
        def scan_group(g, carry):
            FF, mc = carry
            idxl = g * L + lanes
            fv = plsc.load_gather(f_l, [idxl])
            bv = plsc.load_gather(b_l, [idxl])
            m = (idxl < n) & (fv >= c0) & (fv < c0 + cw)
            pos = plsc.cumsum(m.astype(jnp.int32))
            mok = m & (mc + pos - 1 < MCAP)
            pos = plsc.cumsum(mok.astype(jnp.int32))
            dfv = jnp.where(mok, fv - c0, 0)
            bvs = jnp.where(mok, bv, BATCH)
            P = FF + pos - 1
            mdst = jnp.minimum(mc + pos - 1, MCAP - 1)
            plsc.store_scatter(mdf_v, [mdst], dfv, mask=mok)
            plsc.store_scatter(mp_v, [mdst], P, mask=mok)
            plsc.store_scatter(
                bring_v, [(P >> STG_SH) & (NH - 1), P & (STG - 1)],
                bvs, mask=mok)
            cnt = lax.reduce_sum(mok.astype(jnp.int32), axes=(0,))
            return (FF + cnt, mc + cnt)

        return lax.fori_loop(0, ng, scan_group, (F, jnp.int32(0)))

    def flush(h):
        pltpu.async_copy(
            stage_v.at[h], out_hbm.at[bring_v.at[h]], sems
        ).wait()

    def flush_upto(F, H):
        tgt = F >> STG_SH
        for k in range(NH):
            @pl.when(H + k < tgt)
            def _():
                flush((H + k) & (NH - 1))
        return tgt

    def extract_block(buf, row_base, g8, mc):
        def egroup(q, _):
            idxl = q * L + lanes
            m = idxl < mc
            dfv = plsc.load_gather(mdf_v, [idxl])
            dfv = jnp.where(m, dfv, 0)
            P = plsc.load_gather(mp_v, [idxl])
            hv = (P >> STG_SH) & (NH - 1)
            pv = P & (STG - 1)
            for r8 in range(8):
                vals = plsc.load_gather(buf, [zeros + (row_base + r8), dfv])
                plsc.store_scatter(
                    stage_v, [hv, pv, zeros + r8 + g8 * 8], vals, mask=m)
            return 0

        lax.fori_loop(0, (mc + L - 1) // L, egroup, 0)

    def start(j, buf, sem):
        jj = jnp.minimum(j, NBLK - 1)
        c0 = pl.multiple_of(sc_lane0(jj >> 3), SCW)
        r0 = pl.multiple_of((jj & 7) * 8, 8)
        pltpu.async_copy(
            fgT_hbm.at[pl.ds(r0, 8), pl.ds(c0, SCW)], buf, sem)

    def wait(buf, sem):
        pltpu.make_async_copy(
            fgT_hbm.at[pl.ds(0, 8), pl.ds(0, SCW)], buf, sem).wait()

    def after_block(j, buf, F, H, mc):
        extract_block(buf, 0, j & 7, mc)

        def boundary(args):
            FF, HH, _ = args
            HH = flush_upto(FF, HH)
            FF, mcn = phase_a(FF, sc_lane0((j >> 3) + 1), SCW)
            return (FF, HH, mcn)

        return lax.cond(((j & 7) == 7) & (j < NBLK - 1), boundary,
                        lambda a: a, (F, H, mc))

    F, mc = phase_a(jnp.int32(0), sc_lane0(0), SCW)
    H = jnp.int32(0)
    bufs = (buf0, buf1, buf2, buf3)
    bsems = (sem0, sem1, sem2, sem3)
    start(0, buf0, sem0)
    start(1, buf1, sem1)
    start(2, buf2, sem2)

    def pipe(jj, carry):
        F, H, mc = carry
        j0 = 4 * jj
        for t in range(4):
            start(j0 + t + 3, bufs[(t + 3) % 4], bsems[(t + 3) % 4])
            wait(bufs[t], bsems[t])
            F, H, mc = after_block(j0 + t, bufs[t], F, H, mc)
        return (F, H, mc)

    F, H, mc = lax.fori_loop(0, NBLK // 4, pipe, (F, H, mc))
    wait(buf0, sem0)
    wait(buf1, sem1)
    wait(buf2, sem2)

    H = flush_upto(F, H)
    F, mc = phase_a(F, jnp.int32(SP0), SPW)
    for g8 in range(8):
        pltpu.sync_copy(
            fgT_hbm.at[pl.ds(g8 * 8, 8), pl.ds(SP0, SPW)],
            buf0.at[:, pl.ds(0, SPW)])
        extract_block(buf0, 0, g8, mc)

    H = flush_upto(F, H)
    F, mc = phase_a(F, jnp.int32(TAIL0), TAILW)
    pltpu.sync_copy(tail_hbm, tail_v)
    for g8 in range(8):
        extract_block(tail_v, g8 * 8, g8, mc)

    H = flush_upto(F, H)
    resid = F & (STG - 1)

    @pl.when(resid > 0)
    def _():
        h = (F >> STG_SH) & (NH - 1)
        for g in range(STG // L):
            p16 = g * L + lanes
            mpad = p16 >= resid
            plsc.store_scatter(bring_v, [zeros + h, p16], BATCH + p16,
                               mask=mpad)
        flush(h)


def _sc_gather(opT, fgT, tailT):
    mesh = plsc.VectorSubcoreMesh(core_axis_name="c", subcore_axis_name="s")
    cp = pltpu.CompilerParams()
    if "needs_layout_passes" in pltpu.CompilerParams.__dataclass_fields__:
        cp = dataclasses.replace(cp, needs_layout_passes=False)
    cp = dataclasses.replace(cp, use_tc_tiling_on_sc=True)
    k = pl.kernel(
        _sc_body,
        out_type=jax.ShapeDtypeStruct((OUT_PAD, 2 * NUM_REL), jnp.float32),
        mesh=mesh,
        scratch_types=[
            pltpu.VMEM((2, OPB), jnp.int32),
            pltpu.VMEM((LCAP,), jnp.int32),
            pltpu.VMEM((LCAP,), jnp.int32),
            pltpu.VMEM((8, SCW), jnp.float32),
            pltpu.VMEM((8, SCW), jnp.float32),
            pltpu.VMEM((8, SCW), jnp.float32),
            pltpu.VMEM((8, SCW), jnp.float32),
            pltpu.VMEM((NUM_REL, TAILW), jnp.float32),
            pltpu.VMEM((NH, STG, 2 * NUM_REL), jnp.float32),
            pltpu.VMEM((NH, STG), jnp.int32),
            pltpu.VMEM((MCAP,), jnp.int32),
            pltpu.VMEM((MCAP,), jnp.int32),
            pltpu.SemaphoreType.DMA,
            pltpu.SemaphoreType.DMA,
            pltpu.SemaphoreType.DMA,
            pltpu.SemaphoreType.DMA,
            pltpu.SemaphoreType.DMA,
        ],
        compiler_params=cp,
    )
    return k(opT, fgT, tailT)


def _tc_fuse_body(counts_ref, predT_ref, outT_ref):
    c = counts_ref[:, :NUM_REL]
    denom = jnp.sum(c, axis=1, keepdims=True) + EPS
    bias = jnp.log(c / denom + EPS)
    bias = jnp.where(c == 0.0, LOG_PSB, bias)
    col = lax.broadcasted_iota(jnp.int32, c.shape, 1)
    bias = jnp.where(col == 0, LOG_BG, bias)
    outT_ref[...] = predT_ref[...] + bias.T


def _tc_fuse(counts_pad, predT):
    blk = 1024
    grid = BATCH // blk
    outT = pl.pallas_call(
        _tc_fuse_body,
        out_shape=jax.ShapeDtypeStruct((NUM_REL, BATCH), jnp.float32),
        grid=(grid,),
        in_specs=[
            pl.BlockSpec((blk, 2 * NUM_REL), lambda i: (i, 0)),
            pl.BlockSpec((NUM_REL, blk), lambda i: (0, i)),
        ],
        out_specs=pl.BlockSpec((NUM_REL, blk), lambda i: (0, i)),
    )(counts_pad, predT)
    return outT.T


def kernel(pred_dist, gt, obj_pair, fg_count):
    del gt
    fgT = fg_count.T
    tailT = fgT[:, TAIL0:]
    counts_pad = _sc_gather(obj_pair.T, fgT, tailT)
    return _tc_fuse(counts_pad, pred_dist.T)
